# baseline (speedup 1.0000x reference)
import functools

import jax
import jax.numpy as jnp
from jax import lax
from jax.experimental import pallas as pl
from jax.experimental.pallas import tpu as pltpu
from jax.experimental.pallas import tpu_sc as plsc

N = 100000
F = 128
G = 64
EPS = 1e-06

C = 128
NFULL = N // C
TAIL = N - NFULL * C
NW = 32
EXTRA = NFULL % NW
BASE_CPW = NFULL // NW
GRP = 3
NGRP = BASE_CPW // GRP
SLAB = BASE_CPW + 8

R = 10000
NB = N // R


def _sc_body(x_hbm, b_hbm, b2d_hbm, seg_out, xba, xbb, ibig, isc,
             idx_tail, idq, acctab, seg_sh, sem0, sem1):
    cid = lax.axis_index("c")
    sid = lax.axis_index("s")
    w = sid * 2 + cid
    xb = (xba, xbb)
    sem = (sem0, sem1)
    NL = F // 16

    start_w = BASE_CPW * w + jnp.minimum(w, EXTRA)
    has_extra = w < EXTRA

    a_start = (start_w // 8) * 8
    ioff = start_w - a_start
    pltpu.sync_copy(b2d_hbm.at[pl.ds(a_start, SLAB), :], ibig)

    def _z(g, _):
        for l in range(NL):
            acctab[g, pl.ds(16 * l, 16)] = jnp.zeros((16,), jnp.float32)
        return _
    lax.fori_loop(0, G, _z, 0)
    for k in range(G // 16):
        idq[pl.ds(16 * k, 16)] = lax.iota(jnp.int32, 16) + 16 * k
    pltpu.sync_copy(acctab.at[pl.ds(0, 4), :], seg_sh.at[pl.ds(sid * 4, 4), :])
    plsc.subcore_barrier()

    def _valid(s):
        return jnp.logical_or(s < NGRP,
                              jnp.logical_and(s == NGRP, has_extra))

    def _start(s, b):
        @pl.when(s < NGRP)
        def _full():
            ck = start_w + GRP * s
            pltpu.async_copy(x_hbm.at[pl.ds(ck * C, GRP * C), :], xb[b],
                             sem[b])

        @pl.when(jnp.logical_and(s == NGRP, has_extra))
        def _single():
            ck = start_w + BASE_CPW
            pltpu.async_copy(x_hbm.at[pl.ds(ck * C, C), :],
                             xb[b].at[pl.ds(0, C), :], sem[b])

    def _wait(s, b):
        @pl.when(s < NGRP)
        def _full():
            pltpu.make_async_copy(x_hbm.at[pl.ds(0, GRP * C), :], xb[b],
                                  sem[b]).wait()

        @pl.when(jnp.logical_and(s == NGRP, has_extra))
        def _single():
            pltpu.make_async_copy(x_hbm.at[pl.ds(0, C), :],
                                  xb[b].at[pl.ds(0, C), :], sem[b]).wait()

    def _accum(b, nrows, gvec):
        def _acc(r, a):
            return tuple(a[l] + xb[b][r, pl.ds(16 * l, 16)] *
                         xb[b][r, pl.ds(16 * l, 16)] for l in range(NL))
        a = lax.fori_loop(0, nrows, _acc,
                          tuple(jnp.zeros((16,), jnp.float32)
                                for _ in range(NL)))
        g = gvec[0]
        for l in range(NL):
            acctab[g, pl.ds(16 * l, 16)] += a[l]

    def _square(b, nrows):
        def _sq(r, _):
            for l in range(NL):
                v = xb[b][r, pl.ds(16 * l, 16)]
                xb[b][r, pl.ds(16 * l, 16)] = v * v
            return _
        lax.fori_loop(0, nrows, _sq, 0)

    def _scatter_chunk(b, k, ck):
        pltpu.sync_copy(b_hbm.at[pl.ds(ck * C, C)], isc)
        pltpu.sync_copy(xb[b].at[pl.ds(k * C, C), :],
                        seg_sh.at[isc], add=True)

    def _process(s, b):
        @pl.when(s < NGRP)
        def _full():
            lc = GRP * s
            v_first = ibig[ioff + lc, pl.ds(0, 16)]
            v_last = ibig[ioff + lc + (GRP - 1), pl.ds(C - 16, 16)]
            uni = v_first[0] == v_last[15]

            @pl.when(uni)
            def _uniform():
                _accum(b, GRP * C, v_first)

            @pl.when(jnp.logical_not(uni))
            def _boundary():
                _square(b, GRP * C)
                for k in range(GRP):
                    _scatter_chunk(b, k, start_w + lc + k)

        @pl.when(jnp.logical_and(s == NGRP, has_extra))
        def _single():
            lc = BASE_CPW
            v_first = ibig[ioff + lc, pl.ds(0, 16)]
            v_last = ibig[ioff + lc, pl.ds(C - 16, 16)]
            uni = v_first[0] == v_last[15]

            @pl.when(uni)
            def _uniform():
                _accum(b, C, v_first)

            @pl.when(jnp.logical_not(uni))
            def _boundary():
                _square(b, C)
                _scatter_chunk(b, 0, start_w + lc)

    _start(0, 0)
    T = (NGRP + 2) // 2

    def _loop(t, _):
        s0 = 2 * t
        s1 = 2 * t + 1

        @pl.when(_valid(s0))
        def _even():
            _start(s1, 1)
            _wait(s0, 0)
            _process(s0, 0)

        @pl.when(_valid(s1))
        def _odd():
            _start(s1 + 1, 0)
            _wait(s1, 1)
            _process(s1, 1)
        return _
    lax.fori_loop(0, T, _loop, 0)

    pltpu.sync_copy(acctab, seg_sh.at[idq], add=True)

    @pl.when(w == NW - 1)
    def _tail():
        base = NFULL * C
        pltpu.sync_copy(b_hbm.at[pl.ds(base, TAIL)], idx_tail)
        pltpu.sync_copy(x_hbm.at[pl.ds(base, TAIL), :],
                        xba.at[pl.ds(0, TAIL), :])
        def _sq(r, _):
            for l in range(F // 16):
                v = xba[r, pl.ds(16 * l, 16)]
                xba[r, pl.ds(16 * l, 16)] = v * v
            return _
        lax.fori_loop(0, TAIL, _sq, 0)
        pltpu.sync_copy(xba.at[pl.ds(0, TAIL), :],
                        seg_sh.at[idx_tail], add=True)

    plsc.subcore_barrier()

    @pl.when(sid == 0)
    def _emit():
        pltpu.sync_copy(seg_sh, seg_out.at[cid])


@functools.partial(
    pl.kernel,
    out_type=jax.ShapeDtypeStruct((2, G, F), jnp.float32),
    mesh=plsc.VectorSubcoreMesh(core_axis_name="c", subcore_axis_name="s"),
    scratch_types=[
        pltpu.VMEM((GRP * C, F), jnp.float32),
        pltpu.VMEM((GRP * C, F), jnp.float32),
        pltpu.VMEM((SLAB, C), jnp.int32),
        pltpu.VMEM((C,), jnp.int32),
        pltpu.VMEM((TAIL,), jnp.int32),
        pltpu.VMEM((G,), jnp.int32),
        pltpu.VMEM((G, F), jnp.float32),
        pltpu.VMEM_SHARED((G, F), jnp.float32),
        pltpu.SemaphoreType.DMA,
        pltpu.SemaphoreType.DMA,
    ],
)
def _sc_segsum(x_hbm, b_hbm, b2d_hbm, seg_out, *scratch):
    _sc_body(x_hbm, b_hbm, b2d_hbm, seg_out, *scratch)


def _cnt_body(b_ref, out_ref):
    def _step(k, acc):
        b = b_ref[k, 0, :]
        iota_g = jax.lax.broadcasted_iota(jnp.int32, (G, R), 0)
        onehot = (iota_g == b[None, :]).astype(jnp.float32)
        return acc + jnp.sum(onehot, axis=1, keepdims=True)
    cnt = lax.fori_loop(0, NB, _step, jnp.zeros((G, 1), jnp.float32))
    out_ref[...] = jnp.broadcast_to(cnt, (G, F))


def _tc_body(x_ref, b_ref, seg_ref, cnt_ref, w_ref, bias_ref, out_ref,
             inv_ref):
    i = pl.program_id(0)

    @pl.when(i == 0)
    def _mk_inv():
        seg = seg_ref[0] + seg_ref[1]
        mean_sq = seg / jnp.maximum(cnt_ref[...], 1.0)
        inv_ref[...] = jax.lax.rsqrt(mean_sq + EPS)

    b = b_ref[0, 0, :]
    iota_g = jax.lax.broadcasted_iota(jnp.int32, (R, G), 1)
    onehot = (iota_g == b[:, None]).astype(jnp.float32)
    inv = jnp.dot(onehot, inv_ref[...],
                  preferred_element_type=jnp.float32)
    out_ref[...] = x_ref[...] * inv * w_ref[0, :] + bias_ref[0, :]


def kernel(x, batch, weight, bias):
    b32 = batch.astype(jnp.int32)
    b3 = b32.reshape(NB, 1, R)
    npad = ((NFULL + 7) // 8 * 8 + 8) - NFULL
    b2d = jnp.pad(b32[:NFULL * C], (0, npad * C)).reshape(NFULL + npad, C)
    seg = _sc_segsum(x, b32, b2d)
    cnt = pl.pallas_call(
        _cnt_body,
        grid=(1,),
        in_specs=[pl.BlockSpec((NB, 1, R), lambda i: (0, 0, 0))],
        out_specs=pl.BlockSpec((G, F), lambda i: (0, 0)),
        out_shape=jax.ShapeDtypeStruct((G, F), jnp.float32),
    )(b3)
    w2 = weight.reshape(1, F)
    bias2 = bias.reshape(1, F)
    return pl.pallas_call(
        _tc_body,
        grid=(NB,),
        in_specs=[
            pl.BlockSpec((R, F), lambda i: (i, 0)),
            pl.BlockSpec((1, 1, R), lambda i: (i, 0, 0)),
            pl.BlockSpec((2, G, F), lambda i: (0, 0, 0)),
            pl.BlockSpec((G, F), lambda i: (0, 0)),
            pl.BlockSpec((1, F), lambda i: (0, 0)),
            pl.BlockSpec((1, F), lambda i: (0, 0)),
        ],
        out_specs=pl.BlockSpec((R, F), lambda i: (i, 0)),
        out_shape=jax.ShapeDtypeStruct((N, F), jnp.float32),
        scratch_shapes=[pltpu.VMEM((G, F), jnp.float32)],
    )(x, b3, seg, cnt, w2, bias2)


# speedup vs baseline: 1.0789x; 1.0789x over previous
import functools

import jax
import jax.numpy as jnp
from jax import lax
from jax.experimental import pallas as pl
from jax.experimental.pallas import tpu as pltpu
from jax.experimental.pallas import tpu_sc as plsc

N = 100000
F = 128
G = 64
EPS = 1e-06

C = 128
NFULL = N // C
TAIL = N - NFULL * C
NW = 32
EXTRA = NFULL % NW
BASE_CPW = NFULL // NW
GRP = 3
NGRP = BASE_CPW // GRP
SLAB = BASE_CPW + 8

R = 10000
NB = N // R


def _sc_body(x_hbm, b_hbm, b2d_hbm, seg_out, xba, xbb, ibig, isc,
             idx_tail, idq, acctab, seg_sh, sem0, sem1):
    cid = lax.axis_index("c")
    sid = lax.axis_index("s")
    w = sid * 2 + cid
    xb = (xba, xbb)
    sem = (sem0, sem1)
    NL = F // 16

    start_w = BASE_CPW * w + jnp.minimum(w, EXTRA)
    has_extra = w < EXTRA

    a_start = (start_w // 8) * 8
    ioff = start_w - a_start
    pltpu.sync_copy(b2d_hbm.at[pl.ds(a_start, SLAB), :], ibig)

    def _z(g, _):
        for l in range(NL):
            acctab[g, pl.ds(16 * l, 16)] = jnp.zeros((16,), jnp.float32)
        return _
    lax.fori_loop(0, G, _z, 0)
    for k in range(G // 16):
        idq[pl.ds(16 * k, 16)] = lax.iota(jnp.int32, 16) + 16 * k
    pltpu.sync_copy(acctab.at[pl.ds(0, 4), :], seg_sh.at[pl.ds(sid * 4, 4), :])
    plsc.subcore_barrier()

    def _valid(s):
        return jnp.logical_or(s < NGRP,
                              jnp.logical_and(s == NGRP, has_extra))

    def _start(s, b):
        @pl.when(s < NGRP)
        def _full():
            ck = start_w + GRP * s
            pltpu.async_copy(x_hbm.at[pl.ds(ck * C, GRP * C), :], xb[b],
                             sem[b])

        @pl.when(jnp.logical_and(s == NGRP, has_extra))
        def _single():
            ck = start_w + BASE_CPW
            pltpu.async_copy(x_hbm.at[pl.ds(ck * C, C), :],
                             xb[b].at[pl.ds(0, C), :], sem[b])

    def _wait(s, b):
        @pl.when(s < NGRP)
        def _full():
            pltpu.make_async_copy(x_hbm.at[pl.ds(0, GRP * C), :], xb[b],
                                  sem[b]).wait()

        @pl.when(jnp.logical_and(s == NGRP, has_extra))
        def _single():
            pltpu.make_async_copy(x_hbm.at[pl.ds(0, C), :],
                                  xb[b].at[pl.ds(0, C), :], sem[b]).wait()

    def _accum(b, off, nrows, gvec):
        def _acc(r, a):
            return tuple(a[l] + xb[b][off + r, pl.ds(16 * l, 16)] *
                         xb[b][off + r, pl.ds(16 * l, 16)] for l in range(NL))
        a = lax.fori_loop(0, nrows, _acc,
                          tuple(jnp.zeros((16,), jnp.float32)
                                for _ in range(NL)))
        g = gvec[0]
        for l in range(NL):
            acctab[g, pl.ds(16 * l, 16)] += a[l]

    def _square(b, off, nrows):
        def _sq(r, _):
            for l in range(NL):
                v = xb[b][off + r, pl.ds(16 * l, 16)]
                xb[b][off + r, pl.ds(16 * l, 16)] = v * v
            return _
        lax.fori_loop(0, nrows, _sq, 0)

    def _scatter_chunk(b, k, ck):
        pltpu.sync_copy(b_hbm.at[pl.ds(ck * C, C)], isc)
        pltpu.sync_copy(xb[b].at[pl.ds(k * C, C), :],
                        seg_sh.at[isc], add=True)

    def _chunk(b, lc, k):
        v_first = ibig[ioff + lc + k, pl.ds(0, 16)]
        v_last = ibig[ioff + lc + k, pl.ds(C - 16, 16)]
        uni = v_first[0] == v_last[15]

        @pl.when(uni)
        def _uniform():
            _accum(b, k * C, C, v_first)

        @pl.when(jnp.logical_not(uni))
        def _boundary():
            _square(b, k * C, C)
            _scatter_chunk(b, k, start_w + lc + k)

    def _process(s, b):
        @pl.when(s < NGRP)
        def _full():
            lc = GRP * s
            v_first = ibig[ioff + lc, pl.ds(0, 16)]
            v_last = ibig[ioff + lc + (GRP - 1), pl.ds(C - 16, 16)]
            guni = v_first[0] == v_last[15]

            @pl.when(guni)
            def _gu():
                _accum(b, 0, GRP * C, v_first)

            @pl.when(jnp.logical_not(guni))
            def _gb():
                for k in range(GRP):
                    _chunk(b, lc, k)

        @pl.when(jnp.logical_and(s == NGRP, has_extra))
        def _single():
            _chunk(b, BASE_CPW, 0)

    _start(0, 0)
    T = (NGRP + 2) // 2

    def _loop(t, _):
        s0 = 2 * t
        s1 = 2 * t + 1

        @pl.when(_valid(s0))
        def _even():
            _start(s1, 1)
            _wait(s0, 0)
            _process(s0, 0)

        @pl.when(_valid(s1))
        def _odd():
            _start(s1 + 1, 0)
            _wait(s1, 1)
            _process(s1, 1)
        return _
    lax.fori_loop(0, T, _loop, 0)

    pltpu.sync_copy(acctab, seg_sh.at[idq], add=True)

    @pl.when(w == NW - 1)
    def _tail():
        base = NFULL * C
        pltpu.sync_copy(b_hbm.at[pl.ds(base, TAIL)], idx_tail)
        pltpu.sync_copy(x_hbm.at[pl.ds(base, TAIL), :],
                        xba.at[pl.ds(0, TAIL), :])
        def _sq(r, _):
            for l in range(F // 16):
                v = xba[r, pl.ds(16 * l, 16)]
                xba[r, pl.ds(16 * l, 16)] = v * v
            return _
        lax.fori_loop(0, TAIL, _sq, 0)
        pltpu.sync_copy(xba.at[pl.ds(0, TAIL), :],
                        seg_sh.at[idx_tail], add=True)

    plsc.subcore_barrier()

    @pl.when(sid == 0)
    def _emit():
        pltpu.sync_copy(seg_sh, seg_out.at[cid])


@functools.partial(
    pl.kernel,
    out_type=jax.ShapeDtypeStruct((2, G, F), jnp.float32),
    mesh=plsc.VectorSubcoreMesh(core_axis_name="c", subcore_axis_name="s"),
    scratch_types=[
        pltpu.VMEM((GRP * C, F), jnp.float32),
        pltpu.VMEM((GRP * C, F), jnp.float32),
        pltpu.VMEM((SLAB, C), jnp.int32),
        pltpu.VMEM((C,), jnp.int32),
        pltpu.VMEM((TAIL,), jnp.int32),
        pltpu.VMEM((G,), jnp.int32),
        pltpu.VMEM((G, F), jnp.float32),
        pltpu.VMEM_SHARED((G, F), jnp.float32),
        pltpu.SemaphoreType.DMA,
        pltpu.SemaphoreType.DMA,
    ],
)
def _sc_segsum(x_hbm, b_hbm, b2d_hbm, seg_out, *scratch):
    _sc_body(x_hbm, b_hbm, b2d_hbm, seg_out, *scratch)


def _cnt_body(b_ref, out_ref):
    def _step(k, acc):
        b = b_ref[k, 0, :]
        iota_g = jax.lax.broadcasted_iota(jnp.int32, (G, R), 0)
        onehot = (iota_g == b[None, :]).astype(jnp.float32)
        return acc + jnp.sum(onehot, axis=1, keepdims=True)
    cnt = lax.fori_loop(0, NB, _step, jnp.zeros((G, 1), jnp.float32))
    out_ref[...] = jnp.broadcast_to(cnt, (G, F))


def _tc_body(x_ref, b_ref, seg_ref, cnt_ref, w_ref, bias_ref, out_ref,
             inv_ref):
    i = pl.program_id(0)

    @pl.when(i == 0)
    def _mk_inv():
        seg = seg_ref[0] + seg_ref[1]
        mean_sq = seg / jnp.maximum(cnt_ref[...], 1.0)
        inv_ref[...] = jax.lax.rsqrt(mean_sq + EPS)

    b = b_ref[0, 0, :]
    iota_g = jax.lax.broadcasted_iota(jnp.int32, (R, G), 1)
    onehot = (iota_g == b[:, None]).astype(jnp.float32)
    inv = jnp.dot(onehot, inv_ref[...],
                  preferred_element_type=jnp.float32)
    out_ref[...] = x_ref[...] * inv * w_ref[0, :] + bias_ref[0, :]


def kernel(x, batch, weight, bias):
    b32 = batch.astype(jnp.int32)
    b3 = b32.reshape(NB, 1, R)
    npad = ((NFULL + 7) // 8 * 8 + 8) - NFULL
    b2d = jnp.pad(b32[:NFULL * C], (0, npad * C)).reshape(NFULL + npad, C)
    seg = _sc_segsum(x, b32, b2d)
    cnt = pl.pallas_call(
        _cnt_body,
        grid=(1,),
        in_specs=[pl.BlockSpec((NB, 1, R), lambda i: (0, 0, 0))],
        out_specs=pl.BlockSpec((G, F), lambda i: (0, 0)),
        out_shape=jax.ShapeDtypeStruct((G, F), jnp.float32),
    )(b3)
    w2 = weight.reshape(1, F)
    bias2 = bias.reshape(1, F)
    return pl.pallas_call(
        _tc_body,
        grid=(NB,),
        in_specs=[
            pl.BlockSpec((R, F), lambda i: (i, 0)),
            pl.BlockSpec((1, 1, R), lambda i: (i, 0, 0)),
            pl.BlockSpec((2, G, F), lambda i: (0, 0, 0)),
            pl.BlockSpec((G, F), lambda i: (0, 0)),
            pl.BlockSpec((1, F), lambda i: (0, 0)),
            pl.BlockSpec((1, F), lambda i: (0, 0)),
        ],
        out_specs=pl.BlockSpec((R, F), lambda i: (i, 0)),
        out_shape=jax.ShapeDtypeStruct((N, F), jnp.float32),
        scratch_shapes=[pltpu.VMEM((G, F), jnp.float32)],
    )(x, b3, seg, cnt, w2, bias2)


# speedup vs baseline: 1.0841x; 1.0048x over previous
import functools

import jax
import jax.numpy as jnp
from jax import lax
from jax.experimental import pallas as pl
from jax.experimental.pallas import tpu as pltpu
from jax.experimental.pallas import tpu_sc as plsc

N = 100000
F = 128
G = 64
EPS = 1e-06

C = 128
NFULL = N // C
TAIL = N - NFULL * C
NW = 32
EXTRA = NFULL % NW
BASE_CPW = NFULL // NW
GRP = 3
NGRP = BASE_CPW // GRP
SLAB = BASE_CPW + 8

R = 20000
NB = N // R


def _sc_body(x_hbm, b_hbm, b2d_hbm, seg_out, xba, xbb, ibig, isc,
             idx_tail, idq, acctab, seg_sh, sem0, sem1):
    cid = lax.axis_index("c")
    sid = lax.axis_index("s")
    w = sid * 2 + cid
    xb = (xba, xbb)
    sem = (sem0, sem1)
    NL = F // 16

    start_w = BASE_CPW * w + jnp.minimum(w, EXTRA)
    has_extra = w < EXTRA

    a_start = (start_w // 8) * 8
    ioff = start_w - a_start
    pltpu.sync_copy(b2d_hbm.at[pl.ds(a_start, SLAB), :], ibig)

    def _z(g, _):
        for l in range(NL):
            acctab[g, pl.ds(16 * l, 16)] = jnp.zeros((16,), jnp.float32)
        return _
    lax.fori_loop(0, G, _z, 0)
    for k in range(G // 16):
        idq[pl.ds(16 * k, 16)] = lax.iota(jnp.int32, 16) + 16 * k
    pltpu.sync_copy(acctab.at[pl.ds(0, 4), :], seg_sh.at[pl.ds(sid * 4, 4), :])
    plsc.subcore_barrier()

    def _valid(s):
        return jnp.logical_or(s < NGRP,
                              jnp.logical_and(s == NGRP, has_extra))

    def _start(s, b):
        @pl.when(s < NGRP)
        def _full():
            ck = start_w + GRP * s
            pltpu.async_copy(x_hbm.at[pl.ds(ck * C, GRP * C), :], xb[b],
                             sem[b])

        @pl.when(jnp.logical_and(s == NGRP, has_extra))
        def _single():
            ck = start_w + BASE_CPW
            pltpu.async_copy(x_hbm.at[pl.ds(ck * C, C), :],
                             xb[b].at[pl.ds(0, C), :], sem[b])

    def _wait(s, b):
        @pl.when(s < NGRP)
        def _full():
            pltpu.make_async_copy(x_hbm.at[pl.ds(0, GRP * C), :], xb[b],
                                  sem[b]).wait()

        @pl.when(jnp.logical_and(s == NGRP, has_extra))
        def _single():
            pltpu.make_async_copy(x_hbm.at[pl.ds(0, C), :],
                                  xb[b].at[pl.ds(0, C), :], sem[b]).wait()

    def _accum(b, off, nrows, gvec):
        def _acc(r, a):
            return tuple(a[l] + xb[b][off + r, pl.ds(16 * l, 16)] *
                         xb[b][off + r, pl.ds(16 * l, 16)] for l in range(NL))
        a = lax.fori_loop(0, nrows, _acc,
                          tuple(jnp.zeros((16,), jnp.float32)
                                for _ in range(NL)))
        g = gvec[0]
        for l in range(NL):
            acctab[g, pl.ds(16 * l, 16)] += a[l]

    def _square(b, off, nrows):
        def _sq(r, _):
            for l in range(NL):
                v = xb[b][off + r, pl.ds(16 * l, 16)]
                xb[b][off + r, pl.ds(16 * l, 16)] = v * v
            return _
        lax.fori_loop(0, nrows, _sq, 0)

    def _scatter_chunk(b, k, ck):
        pltpu.sync_copy(b_hbm.at[pl.ds(ck * C, C)], isc)
        pltpu.sync_copy(xb[b].at[pl.ds(k * C, C), :],
                        seg_sh.at[isc], add=True)

    def _chunk(b, lc, k):
        v_first = ibig[ioff + lc + k, pl.ds(0, 16)]
        v_last = ibig[ioff + lc + k, pl.ds(C - 16, 16)]
        uni = v_first[0] == v_last[15]

        @pl.when(uni)
        def _uniform():
            _accum(b, k * C, C, v_first)

        @pl.when(jnp.logical_not(uni))
        def _boundary():
            _square(b, k * C, C)
            _scatter_chunk(b, k, start_w + lc + k)

    def _process(s, b):
        @pl.when(s < NGRP)
        def _full():
            lc = GRP * s
            v_first = ibig[ioff + lc, pl.ds(0, 16)]
            v_last = ibig[ioff + lc + (GRP - 1), pl.ds(C - 16, 16)]
            guni = v_first[0] == v_last[15]

            @pl.when(guni)
            def _gu():
                _accum(b, 0, GRP * C, v_first)

            @pl.when(jnp.logical_not(guni))
            def _gb():
                for k in range(GRP):
                    _chunk(b, lc, k)

        @pl.when(jnp.logical_and(s == NGRP, has_extra))
        def _single():
            _chunk(b, BASE_CPW, 0)

    _start(0, 0)
    T = (NGRP + 2) // 2

    def _loop(t, _):
        s0 = 2 * t
        s1 = 2 * t + 1

        @pl.when(_valid(s0))
        def _even():
            _start(s1, 1)
            _wait(s0, 0)
            _process(s0, 0)

        @pl.when(_valid(s1))
        def _odd():
            _start(s1 + 1, 0)
            _wait(s1, 1)
            _process(s1, 1)
        return _
    lax.fori_loop(0, T, _loop, 0)

    pltpu.sync_copy(acctab, seg_sh.at[idq], add=True)

    @pl.when(w == NW - 1)
    def _tail():
        base = NFULL * C
        pltpu.sync_copy(b_hbm.at[pl.ds(base, TAIL)], idx_tail)
        pltpu.sync_copy(x_hbm.at[pl.ds(base, TAIL), :],
                        xba.at[pl.ds(0, TAIL), :])
        def _sq(r, _):
            for l in range(F // 16):
                v = xba[r, pl.ds(16 * l, 16)]
                xba[r, pl.ds(16 * l, 16)] = v * v
            return _
        lax.fori_loop(0, TAIL, _sq, 0)
        pltpu.sync_copy(xba.at[pl.ds(0, TAIL), :],
                        seg_sh.at[idx_tail], add=True)

    plsc.subcore_barrier()

    @pl.when(sid == 0)
    def _emit():
        pltpu.sync_copy(seg_sh, seg_out.at[cid])


@functools.partial(
    pl.kernel,
    out_type=jax.ShapeDtypeStruct((2, G, F), jnp.float32),
    mesh=plsc.VectorSubcoreMesh(core_axis_name="c", subcore_axis_name="s"),
    scratch_types=[
        pltpu.VMEM((GRP * C, F), jnp.float32),
        pltpu.VMEM((GRP * C, F), jnp.float32),
        pltpu.VMEM((SLAB, C), jnp.int32),
        pltpu.VMEM((C,), jnp.int32),
        pltpu.VMEM((TAIL,), jnp.int32),
        pltpu.VMEM((G,), jnp.int32),
        pltpu.VMEM((G, F), jnp.float32),
        pltpu.VMEM_SHARED((G, F), jnp.float32),
        pltpu.SemaphoreType.DMA,
        pltpu.SemaphoreType.DMA,
    ],
)
def _sc_segsum(x_hbm, b_hbm, b2d_hbm, seg_out, *scratch):
    _sc_body(x_hbm, b_hbm, b2d_hbm, seg_out, *scratch)


def _cnt_body(b_ref, out_ref):
    def _step(k, acc):
        b = b_ref[k, 0, :]
        iota_g = jax.lax.broadcasted_iota(jnp.int32, (G, R), 0)
        onehot = (iota_g == b[None, :]).astype(jnp.float32)
        return acc + jnp.sum(onehot, axis=1, keepdims=True)
    cnt = lax.fori_loop(0, NB, _step, jnp.zeros((G, 1), jnp.float32))
    out_ref[...] = jnp.broadcast_to(cnt, (G, F))


def _tc_body(x_ref, b_ref, seg_ref, cnt_ref, w_ref, bias_ref, out_ref,
             inv_ref):
    i = pl.program_id(0)

    @pl.when(i == 0)
    def _mk_inv():
        seg = seg_ref[0] + seg_ref[1]
        mean_sq = seg / jnp.maximum(cnt_ref[...], 1.0)
        inv_ref[...] = jax.lax.rsqrt(mean_sq + EPS)

    b = b_ref[0, 0, :]
    iota_g = jax.lax.broadcasted_iota(jnp.int32, (R, G), 1)
    onehot = (iota_g == b[:, None]).astype(jnp.float32)
    inv = jnp.dot(onehot, inv_ref[...],
                  preferred_element_type=jnp.float32)
    out_ref[...] = x_ref[...] * inv * w_ref[0, :] + bias_ref[0, :]


def kernel(x, batch, weight, bias):
    b32 = batch.astype(jnp.int32)
    b3 = b32.reshape(NB, 1, R)
    npad = ((NFULL + 7) // 8 * 8 + 8) - NFULL
    b2d = jnp.pad(b32[:NFULL * C], (0, npad * C)).reshape(NFULL + npad, C)
    seg = _sc_segsum(x, b32, b2d)
    cnt = pl.pallas_call(
        _cnt_body,
        grid=(1,),
        in_specs=[pl.BlockSpec((NB, 1, R), lambda i: (0, 0, 0))],
        out_specs=pl.BlockSpec((G, F), lambda i: (0, 0)),
        out_shape=jax.ShapeDtypeStruct((G, F), jnp.float32),
    )(b3)
    w2 = weight.reshape(1, F)
    bias2 = bias.reshape(1, F)
    return pl.pallas_call(
        _tc_body,
        grid=(NB,),
        in_specs=[
            pl.BlockSpec((R, F), lambda i: (i, 0)),
            pl.BlockSpec((1, 1, R), lambda i: (i, 0, 0)),
            pl.BlockSpec((2, G, F), lambda i: (0, 0, 0)),
            pl.BlockSpec((G, F), lambda i: (0, 0)),
            pl.BlockSpec((1, F), lambda i: (0, 0)),
            pl.BlockSpec((1, F), lambda i: (0, 0)),
        ],
        out_specs=pl.BlockSpec((R, F), lambda i: (i, 0)),
        out_shape=jax.ShapeDtypeStruct((N, F), jnp.float32),
        scratch_shapes=[pltpu.VMEM((G, F), jnp.float32)],
    )(x, b3, seg, cnt, w2, bias2)
